# baseline (speedup 1.0000x reference)
import functools

import jax
import jax.numpy as jnp
from jax import lax
from jax.experimental import pallas as pl
from jax.experimental.pallas import tpu as pltpu
from jax.experimental.pallas import tpu_sc as plsc


def _bar(v):
    return jax.lax.bitcast_convert_type(
        jax.lax.bitcast_convert_type(v, jnp.int32) + jnp.int32(0), jnp.float32)


def _bdot(a, b):
    return jnp.dot(a.astype(jnp.bfloat16), b.astype(jnp.bfloat16),
                   preferred_element_type=jnp.float32)


def _k1_body(x1_ref, x2t_ref, gidx_ref, w_ref, *, n2):
    x1 = x1_ref[0]
    x2t = x2t_ref[0]
    tq = x1.shape[0]

    ab = _bdot(x1, x2t)
    x1s = _bar(x1 * x1)
    a2 = _bar(_bar(x1s[:, 0:1] + x1s[:, 1:2]) + x1s[:, 2:3])
    x2s = _bar(x2t * x2t)
    b2 = _bar(_bar(x2s[0:1, :] + x2s[1:2, :]) + x2s[2:3, :])
    dist = _bar(_bar(_bar(-2.0 * ab) + a2) + b2)

    iota = jax.lax.broadcasted_iota(jnp.int32, (tq, n2), 1)
    d = dist
    idxs = []
    rs = []
    rsum = jnp.zeros((tq, 1), jnp.float32)
    for _ in range(3):
        m = jnp.min(d, axis=1, keepdims=True)
        idx = jnp.min(jnp.where(d == m, iota, n2), axis=1, keepdims=True)
        onehot = iota == idx
        r = 1.0 / (m + 1e-8)
        idxs.append(idx)
        rs.append(r)
        rsum = rsum + r
        d = jnp.where(onehot, jnp.inf, d)

    gidx_ref[0] = jnp.concatenate(idxs, axis=1) + pl.program_id(0) * n2
    ws = [jnp.broadcast_to(r / rsum, (tq, 16)) for r in rs]
    w_ref[0] = jnp.concatenate(ws, axis=1)


def _k1b_body(f1_ref, it_ref, w0at_ref, w0bt_ref, b0_ref,
              h1_ref, s1_ref, ss1_ref):
    h1 = (_bdot(f1_ref[0], w0at_ref[...])
          + _bdot(it_ref[0], w0bt_ref[...])
          + b0_ref[...])
    h1_ref[0] = h1

    @pl.when((pl.program_id(0) == 0) & (pl.program_id(1) == 0))
    def _():
        s1_ref[...] = jnp.zeros_like(s1_ref)
        ss1_ref[...] = jnp.zeros_like(ss1_ref)

    s1_ref[...] += jnp.sum(h1, axis=0, keepdims=True)
    ss1_ref[...] += jnp.sum(h1 * h1, axis=0, keepdims=True)


def _make_sc_interp(Q, V, C2):
    info = plsc.get_sparse_core_info()
    nw = info.num_cores * info.num_subcores
    qpw = Q // nw
    CQ = 16
    nch = qpw // CQ
    mesh = plsc.VectorSubcoreMesh(core_axis_name="c", subcore_axis_name="s")

    @functools.partial(
        pl.kernel, mesh=mesh,
        out_type=jax.ShapeDtypeStruct((Q, C2), jnp.float32),
        scratch_types=[
            pltpu.VMEM((CQ * 3,), jnp.int32),
            pltpu.VMEM((CQ * 48,), jnp.float32),
            pltpu.VMEM((CQ * 3, C2), jnp.float32),
            pltpu.VMEM((CQ, C2), jnp.float32),
            pltpu.SemaphoreType.DMA,
        ],
    )
    def sc_interp(f2_hbm, idx_hbm, w_hbm, out_hbm, idx_v, w_v, rows_v, out_v, sem):
        wid = lax.axis_index("s") * info.num_cores + lax.axis_index("c")
        qbase = wid * qpw

        def chunk(ci, carry):
            q0 = qbase + ci * CQ
            pltpu.sync_copy(idx_hbm.at[pl.ds(q0 * 3, CQ * 3)], idx_v)
            pltpu.sync_copy(w_hbm.at[pl.ds(q0 * 48, CQ * 48)], w_v)
            pltpu.async_copy(f2_hbm.at[idx_v], rows_v, sem).wait()
            for q in range(CQ):
                wv = [w_v[pl.ds(q * 48 + 16 * j, 16)] for j in range(3)]
                for c in range(C2 // 16):
                    acc = wv[0] * rows_v[3 * q, pl.ds(c * 16, 16)]
                    acc = acc + wv[1] * rows_v[3 * q + 1, pl.ds(c * 16, 16)]
                    acc = acc + wv[2] * rows_v[3 * q + 2, pl.ds(c * 16, 16)]
                    out_v[q, pl.ds(c * 16, 16)] = acc
            pltpu.sync_copy(out_v, out_hbm.at[pl.ds(q0, CQ)])
            return carry

        lax.fori_loop(0, nch, chunk, 0)

    return sc_interp


def _k2_body(h1_ref, s1_ref, ss1_ref, g0_ref, bt0_ref, w1t_ref, b1_ref,
             h2_ref, s2_ref, ss2_ref, *, count):
    mean = s1_ref[...] / count
    var = ss1_ref[...] / count - mean * mean
    rstd = jax.lax.rsqrt(var + 1e-5)
    a1 = jnp.maximum((h1_ref[0] - mean) * (rstd * g0_ref[...]) + bt0_ref[...],
                     0.0)
    h2 = _bdot(a1, w1t_ref[...]) + b1_ref[...]
    h2_ref[0] = h2

    @pl.when((pl.program_id(0) == 0) & (pl.program_id(1) == 0))
    def _():
        s2_ref[...] = jnp.zeros_like(s2_ref)
        ss2_ref[...] = jnp.zeros_like(ss2_ref)

    s2_ref[...] += jnp.sum(h2, axis=0, keepdims=True)
    ss2_ref[...] += jnp.sum(h2 * h2, axis=0, keepdims=True)


def _k3_body(h2_ref, s2_ref, ss2_ref, g1_ref, bt1_ref, out_ref, *, count):
    mean = s2_ref[...] / count
    var = ss2_ref[...] / count - mean * mean
    rstd = jax.lax.rsqrt(var + 1e-5)
    out_ref[0] = jnp.maximum(
        (h2_ref[0] - mean) * (rstd * g1_ref[...]) + bt1_ref[...], 0.0)


def kernel(x1, x2, feaure1, feature2, W0, b0, gamma0, beta0, W1, b1, gamma1, beta1):
    B, N1, _ = x1.shape
    N2 = x2.shape[1]
    C1 = feaure1.shape[-1]
    C2 = feature2.shape[-1]
    H1 = W0.shape[0]
    H2 = W1.shape[0]
    count = float(B * N1)

    TQ1 = min(256, N1)
    TQ2 = min(512, N1)
    TQ3 = min(1024, N1)
    nb1 = N1 // TQ1
    nb2 = N1 // TQ2
    nb3 = N1 // TQ3

    x2t = jnp.swapaxes(x2, 1, 2)
    w0t = jnp.transpose(W0)
    w0at, w0bt = w0t[:C1], w0t[C1:]
    w1t = jnp.transpose(W1)
    b0r = b0.reshape(1, H1)
    g0r = gamma0.reshape(1, H1)
    bt0r = beta0.reshape(1, H1)
    b1r = b1.reshape(1, H2)
    g1r = gamma1.reshape(1, H2)
    bt1r = beta1.reshape(1, H2)

    rep = lambda shape: pl.BlockSpec(shape, lambda b, i: (0,) * len(shape))
    per_b = lambda shape: pl.BlockSpec(shape, lambda b, i: (b, 0, 0))
    tiled = lambda shape: pl.BlockSpec(shape, lambda b, i: (b, i, 0))

    gidx, wgt = pl.pallas_call(
        functools.partial(_k1_body, n2=N2),
        grid=(B, nb1),
        in_specs=[
            tiled((1, TQ1, 3)),
            per_b((1, 3, N2)),
        ],
        out_specs=[
            tiled((1, TQ1, 3)),
            tiled((1, TQ1, 48)),
        ],
        out_shape=[
            jax.ShapeDtypeStruct((B, N1, 3), jnp.int32),
            jax.ShapeDtypeStruct((B, N1, 48), jnp.float32),
        ],
    )(x1, x2t)

    interp = _make_sc_interp(B * N1, B * N2, C2)(
        feature2.reshape(B * N2, C2),
        gidx.reshape(B * N1 * 3),
        wgt.reshape(B * N1 * 48),
    ).reshape(B, N1, C2)

    h1, s1, ss1 = pl.pallas_call(
        _k1b_body,
        grid=(B, nb1),
        in_specs=[
            tiled((1, TQ1, C1)),
            tiled((1, TQ1, C2)),
            rep((C1, H1)),
            rep((C2, H1)),
            rep((1, H1)),
        ],
        out_specs=[
            tiled((1, TQ1, H1)),
            rep((1, H1)),
            rep((1, H1)),
        ],
        out_shape=[
            jax.ShapeDtypeStruct((B, N1, H1), jnp.float32),
            jax.ShapeDtypeStruct((1, H1), jnp.float32),
            jax.ShapeDtypeStruct((1, H1), jnp.float32),
        ],
    )(feaure1, interp, w0at, w0bt, b0r)

    h2, s2, ss2 = pl.pallas_call(
        functools.partial(_k2_body, count=count),
        grid=(B, nb2),
        in_specs=[
            tiled((1, TQ2, H1)),
            rep((1, H1)),
            rep((1, H1)),
            rep((1, H1)),
            rep((1, H1)),
            rep((H1, H2)),
            rep((1, H2)),
        ],
        out_specs=[
            tiled((1, TQ2, H2)),
            rep((1, H2)),
            rep((1, H2)),
        ],
        out_shape=[
            jax.ShapeDtypeStruct((B, N1, H2), jnp.float32),
            jax.ShapeDtypeStruct((1, H2), jnp.float32),
            jax.ShapeDtypeStruct((1, H2), jnp.float32),
        ],
    )(h1, s1, ss1, g0r, bt0r, w1t, b1r)

    out = pl.pallas_call(
        functools.partial(_k3_body, count=count),
        grid=(B, nb3),
        in_specs=[
            tiled((1, TQ3, H2)),
            rep((1, H2)),
            rep((1, H2)),
            rep((1, H2)),
            rep((1, H2)),
        ],
        out_specs=tiled((1, TQ3, H2)),
        out_shape=jax.ShapeDtypeStruct((B, N1, H2), jnp.float32),
    )(h2, s2, ss2, g1r, bt1r)

    return out


# speedup vs baseline: 1.1734x; 1.1734x over previous
import functools

import jax
import jax.numpy as jnp
from jax import lax
from jax.experimental import pallas as pl
from jax.experimental.pallas import tpu as pltpu
from jax.experimental.pallas import tpu_sc as plsc


def _bar(v):
    return jax.lax.bitcast_convert_type(
        jax.lax.bitcast_convert_type(v, jnp.int32) + jnp.int32(0), jnp.float32)


def _bdot(a, b):
    return jnp.dot(a.astype(jnp.bfloat16), b.astype(jnp.bfloat16),
                   preferred_element_type=jnp.float32)


def _k1_body(x1_ref, x2t_ref, gidx_ref, w_ref, *, n2):
    x1 = x1_ref[0]
    x2t = x2t_ref[0]
    tq = x1.shape[0]

    ab = _bdot(x1, x2t)
    x1s = _bar(x1 * x1)
    a2 = _bar(_bar(x1s[:, 0:1] + x1s[:, 1:2]) + x1s[:, 2:3])
    x2s = _bar(x2t * x2t)
    b2 = _bar(_bar(x2s[0:1, :] + x2s[1:2, :]) + x2s[2:3, :])
    dist = _bar(_bar(_bar(-2.0 * ab) + a2) + b2)

    iota = jax.lax.broadcasted_iota(jnp.int32, (tq, n2), 1)
    d = dist
    idxs = []
    rs = []
    rsum = jnp.zeros((tq, 1), jnp.float32)
    for _ in range(3):
        m = jnp.min(d, axis=1, keepdims=True)
        idx = jnp.min(jnp.where(d == m, iota, n2), axis=1, keepdims=True)
        onehot = iota == idx
        r = 1.0 / (m + 1e-8)
        idxs.append(idx)
        rs.append(r)
        rsum = rsum + r
        d = jnp.where(onehot, jnp.inf, d)

    gidx_ref[0] = jnp.concatenate(idxs, axis=1) + pl.program_id(0) * n2
    ws = [jnp.broadcast_to(r / rsum, (tq, 16)) for r in rs]
    w_ref[0] = jnp.concatenate(ws, axis=1)


def _k1b_body(f1_ref, it_ref, w0at_ref, w0bt_ref, b0_ref,
              h1_ref, s1_ref, ss1_ref):
    h1 = (_bdot(f1_ref[0], w0at_ref[...])
          + _bdot(it_ref[0], w0bt_ref[...])
          + b0_ref[...])
    h1_ref[0] = h1

    @pl.when((pl.program_id(0) == 0) & (pl.program_id(1) == 0))
    def _():
        s1_ref[...] = jnp.zeros_like(s1_ref)
        ss1_ref[...] = jnp.zeros_like(ss1_ref)

    s1_ref[...] += jnp.sum(h1, axis=0, keepdims=True)
    ss1_ref[...] += jnp.sum(h1 * h1, axis=0, keepdims=True)


def _make_sc_interp(Q, V, C2):
    info = plsc.get_sparse_core_info()
    nw = info.num_cores * info.num_subcores
    qpw = Q // nw
    CQ = 16
    nch = qpw // CQ
    npair = nch // 2
    mesh = plsc.VectorSubcoreMesh(core_axis_name="c", subcore_axis_name="s")

    @functools.partial(
        pl.kernel, mesh=mesh,
        out_type=jax.ShapeDtypeStruct((Q, C2), jnp.float32),
        scratch_types=[
            pltpu.VMEM((qpw * 3,), jnp.int32),
            pltpu.VMEM((2 * CQ * 48,), jnp.float32),
            pltpu.VMEM((CQ * 3, C2), jnp.float32),
            pltpu.VMEM((CQ * 3, C2), jnp.float32),
            pltpu.VMEM((CQ, C2), jnp.float32),
            pltpu.SemaphoreType.DMA,
            pltpu.SemaphoreType.DMA,
        ],
    )
    def sc_interp(f2_hbm, idx_hbm, w_hbm, out_hbm, idx_v, w_v, rows_a, rows_b,
                  out_v, sem_a, sem_b):
        wid = lax.axis_index("s") * info.num_cores + lax.axis_index("c")
        qbase = wid * qpw
        pltpu.sync_copy(idx_hbm.at[pl.ds(qbase * 3, qpw * 3)], idx_v)

        def start(ci, buf, sem):
            return pltpu.async_copy(
                f2_hbm.at[idx_v.at[pl.ds(ci * (CQ * 3), CQ * 3)]], buf, sem)

        def compute(buf, woff, ci):
            for q in range(CQ):
                wv = [w_v[pl.ds(woff + q * 48 + 16 * j, 16)] for j in range(3)]
                for c in range(C2 // 16):
                    acc = wv[0] * buf[3 * q, pl.ds(c * 16, 16)]
                    acc = acc + wv[1] * buf[3 * q + 1, pl.ds(c * 16, 16)]
                    acc = acc + wv[2] * buf[3 * q + 2, pl.ds(c * 16, 16)]
                    out_v[q, pl.ds(c * 16, 16)] = acc
            pltpu.sync_copy(out_v, out_hbm.at[pl.ds(qbase + ci * CQ, CQ)])

        start(0, rows_a, sem_a)
        start(1, rows_b, sem_b)

        def pair(k, carry):
            c0 = 2 * k
            pltpu.sync_copy(
                w_hbm.at[pl.ds((qbase + c0 * CQ) * 48, 2 * CQ * 48)], w_v)
            pltpu.make_async_copy(
                f2_hbm.at[idx_v.at[pl.ds(0, CQ * 3)]], rows_a, sem_a).wait()
            compute(rows_a, 0, c0)

            @pl.when(k + 1 < npair)
            def _():
                start(c0 + 2, rows_a, sem_a)

            pltpu.make_async_copy(
                f2_hbm.at[idx_v.at[pl.ds(0, CQ * 3)]], rows_b, sem_b).wait()
            compute(rows_b, CQ * 48, c0 + 1)

            @pl.when(k + 1 < npair)
            def _():
                start(c0 + 3, rows_b, sem_b)

            return carry

        lax.fori_loop(0, npair, pair, 0)

    return sc_interp


def _k2_body(h1_ref, s1_ref, ss1_ref, g0_ref, bt0_ref, w1t_ref, b1_ref,
             h2_ref, s2_ref, ss2_ref, *, count):
    mean = s1_ref[...] / count
    var = ss1_ref[...] / count - mean * mean
    rstd = jax.lax.rsqrt(var + 1e-5)
    a1 = jnp.maximum((h1_ref[0] - mean) * (rstd * g0_ref[...]) + bt0_ref[...],
                     0.0)
    h2 = _bdot(a1, w1t_ref[...]) + b1_ref[...]
    h2_ref[0] = h2

    @pl.when((pl.program_id(0) == 0) & (pl.program_id(1) == 0))
    def _():
        s2_ref[...] = jnp.zeros_like(s2_ref)
        ss2_ref[...] = jnp.zeros_like(ss2_ref)

    s2_ref[...] += jnp.sum(h2, axis=0, keepdims=True)
    ss2_ref[...] += jnp.sum(h2 * h2, axis=0, keepdims=True)


def _k3_body(h2_ref, s2_ref, ss2_ref, g1_ref, bt1_ref, out_ref, *, count):
    mean = s2_ref[...] / count
    var = ss2_ref[...] / count - mean * mean
    rstd = jax.lax.rsqrt(var + 1e-5)
    out_ref[0] = jnp.maximum(
        (h2_ref[0] - mean) * (rstd * g1_ref[...]) + bt1_ref[...], 0.0)


def kernel(x1, x2, feaure1, feature2, W0, b0, gamma0, beta0, W1, b1, gamma1, beta1):
    B, N1, _ = x1.shape
    N2 = x2.shape[1]
    C1 = feaure1.shape[-1]
    C2 = feature2.shape[-1]
    H1 = W0.shape[0]
    H2 = W1.shape[0]
    count = float(B * N1)

    TQ1 = min(256, N1)
    TQ2 = min(512, N1)
    TQ3 = min(1024, N1)
    nb1 = N1 // TQ1
    nb2 = N1 // TQ2
    nb3 = N1 // TQ3

    x2t = jnp.swapaxes(x2, 1, 2)
    w0t = jnp.transpose(W0)
    w0at, w0bt = w0t[:C1], w0t[C1:]
    w1t = jnp.transpose(W1)
    b0r = b0.reshape(1, H1)
    g0r = gamma0.reshape(1, H1)
    bt0r = beta0.reshape(1, H1)
    b1r = b1.reshape(1, H2)
    g1r = gamma1.reshape(1, H2)
    bt1r = beta1.reshape(1, H2)

    rep = lambda shape: pl.BlockSpec(shape, lambda b, i: (0,) * len(shape))
    per_b = lambda shape: pl.BlockSpec(shape, lambda b, i: (b, 0, 0))
    tiled = lambda shape: pl.BlockSpec(shape, lambda b, i: (b, i, 0))

    gidx, wgt = pl.pallas_call(
        functools.partial(_k1_body, n2=N2),
        grid=(B, nb1),
        in_specs=[
            tiled((1, TQ1, 3)),
            per_b((1, 3, N2)),
        ],
        out_specs=[
            tiled((1, TQ1, 3)),
            tiled((1, TQ1, 48)),
        ],
        out_shape=[
            jax.ShapeDtypeStruct((B, N1, 3), jnp.int32),
            jax.ShapeDtypeStruct((B, N1, 48), jnp.float32),
        ],
    )(x1, x2t)

    interp = _make_sc_interp(B * N1, B * N2, C2)(
        feature2.reshape(B * N2, C2),
        gidx.reshape(B * N1 * 3),
        wgt.reshape(B * N1 * 48),
    ).reshape(B, N1, C2)

    h1, s1, ss1 = pl.pallas_call(
        _k1b_body,
        grid=(B, nb1),
        in_specs=[
            tiled((1, TQ1, C1)),
            tiled((1, TQ1, C2)),
            rep((C1, H1)),
            rep((C2, H1)),
            rep((1, H1)),
        ],
        out_specs=[
            tiled((1, TQ1, H1)),
            rep((1, H1)),
            rep((1, H1)),
        ],
        out_shape=[
            jax.ShapeDtypeStruct((B, N1, H1), jnp.float32),
            jax.ShapeDtypeStruct((1, H1), jnp.float32),
            jax.ShapeDtypeStruct((1, H1), jnp.float32),
        ],
    )(feaure1, interp, w0at, w0bt, b0r)

    h2, s2, ss2 = pl.pallas_call(
        functools.partial(_k2_body, count=count),
        grid=(B, nb2),
        in_specs=[
            tiled((1, TQ2, H1)),
            rep((1, H1)),
            rep((1, H1)),
            rep((1, H1)),
            rep((1, H1)),
            rep((H1, H2)),
            rep((1, H2)),
        ],
        out_specs=[
            tiled((1, TQ2, H2)),
            rep((1, H2)),
            rep((1, H2)),
        ],
        out_shape=[
            jax.ShapeDtypeStruct((B, N1, H2), jnp.float32),
            jax.ShapeDtypeStruct((1, H2), jnp.float32),
            jax.ShapeDtypeStruct((1, H2), jnp.float32),
        ],
    )(h1, s1, ss1, g0r, bt0r, w1t, b1r)

    out = pl.pallas_call(
        functools.partial(_k3_body, count=count),
        grid=(B, nb3),
        in_specs=[
            tiled((1, TQ3, H2)),
            rep((1, H2)),
            rep((1, H2)),
            rep((1, H2)),
            rep((1, H2)),
        ],
        out_specs=tiled((1, TQ3, H2)),
        out_shape=jax.ShapeDtypeStruct((B, N1, H2), jnp.float32),
    )(h2, s2, ss2, g1r, bt1r)

    return out


# speedup vs baseline: 1.3376x; 1.1399x over previous
import functools

import jax
import jax.numpy as jnp
from jax import lax
from jax.experimental import pallas as pl
from jax.experimental.pallas import tpu as pltpu
from jax.experimental.pallas import tpu_sc as plsc


def _bar(v):
    return jax.lax.bitcast_convert_type(
        jax.lax.bitcast_convert_type(v, jnp.int32) + jnp.int32(0), jnp.float32)


def _bdot(a, b):
    return jnp.dot(a.astype(jnp.bfloat16), b.astype(jnp.bfloat16),
                   preferred_element_type=jnp.float32)


def _k1_body(x1_ref, x2t_ref, gidx_ref, w_ref, *, n2):
    x1 = x1_ref[0]
    x2t = x2t_ref[0]
    tq = x1.shape[0]

    ab = _bdot(x1, x2t)
    x1s = _bar(x1 * x1)
    a2 = _bar(_bar(x1s[:, 0:1] + x1s[:, 1:2]) + x1s[:, 2:3])
    x2s = _bar(x2t * x2t)
    b2 = _bar(_bar(x2s[0:1, :] + x2s[1:2, :]) + x2s[2:3, :])
    dist = _bar(_bar(_bar(-2.0 * ab) + a2) + b2)

    iota = jax.lax.broadcasted_iota(jnp.int32, (tq, n2), 1)
    d = dist
    idxs = []
    rs = []
    rsum = jnp.zeros((tq, 1), jnp.float32)
    for _ in range(3):
        m = jnp.min(d, axis=1, keepdims=True)
        idx = jnp.min(jnp.where(d == m, iota, n2), axis=1, keepdims=True)
        onehot = iota == idx
        r = 1.0 / (m + 1e-8)
        idxs.append(idx)
        rs.append(r)
        rsum = rsum + r
        d = jnp.where(onehot, jnp.inf, d)

    gidx_ref[0] = jnp.concatenate(idxs, axis=1) + pl.program_id(0) * n2
    ws = [jnp.broadcast_to(r / rsum, (tq, 16)) for r in rs]
    w_ref[0] = jnp.concatenate(ws, axis=1)


def _k1b_body(f1_ref, it_ref, w0at_ref, w0bt_ref, b0_ref,
              h1_ref, s1_ref, ss1_ref):
    h1 = (_bdot(f1_ref[0], w0at_ref[...])
          + _bdot(it_ref[0], w0bt_ref[...])
          + b0_ref[...])
    h1_ref[0] = h1

    @pl.when((pl.program_id(0) == 0) & (pl.program_id(1) == 0))
    def _():
        s1_ref[...] = jnp.zeros_like(s1_ref)
        ss1_ref[...] = jnp.zeros_like(ss1_ref)

    s1_ref[...] += jnp.sum(h1, axis=0, keepdims=True)
    ss1_ref[...] += jnp.sum(h1 * h1, axis=0, keepdims=True)


def _make_sc_interp(Q, V, C2):
    info = plsc.get_sparse_core_info()
    nw = info.num_cores * info.num_subcores
    qpw = Q // nw
    CQ = 16
    nch = qpw // CQ
    npair = nch // 2
    mesh = plsc.VectorSubcoreMesh(core_axis_name="c", subcore_axis_name="s")

    @functools.partial(
        pl.kernel, mesh=mesh,
        out_type=jax.ShapeDtypeStruct((Q, C2), jnp.float32),
        scratch_types=[
            pltpu.VMEM((qpw * 3,), jnp.int32),
            pltpu.VMEM((8 * CQ * 48,), jnp.float32),
            pltpu.VMEM((CQ * 3, C2), jnp.float32),
            pltpu.VMEM((CQ * 3, C2), jnp.float32),
            pltpu.VMEM((CQ, C2), jnp.float32),
            pltpu.VMEM((CQ, C2), jnp.float32),
            pltpu.SemaphoreType.DMA,
            pltpu.SemaphoreType.DMA,
            pltpu.SemaphoreType.DMA,
            pltpu.SemaphoreType.DMA,
        ],
    )
    def sc_interp(f2_hbm, idx_hbm, w_hbm, out_hbm, idx_v, w_v, rows_a, rows_b,
                  out_a, out_b, sem_a, sem_b, osem_a, osem_b):
        wid = lax.axis_index("s") * info.num_cores + lax.axis_index("c")
        qbase = wid * qpw
        pltpu.sync_copy(idx_hbm.at[pl.ds(qbase * 3, qpw * 3)], idx_v)

        def start(ci, buf, sem):
            return pltpu.async_copy(
                f2_hbm.at[idx_v.at[pl.ds(ci * (CQ * 3), CQ * 3)]], buf, sem)

        def compute(buf, out_v, woff, ci):
            for q in range(CQ):
                wv = [w_v[pl.ds(woff + q * 48 + 16 * j, 16)] for j in range(3)]
                for c in range(C2 // 16):
                    acc = wv[0] * buf[3 * q, pl.ds(c * 16, 16)]
                    acc = acc + wv[1] * buf[3 * q + 1, pl.ds(c * 16, 16)]
                    acc = acc + wv[2] * buf[3 * q + 2, pl.ds(c * 16, 16)]
                    out_v[q, pl.ds(c * 16, 16)] = acc

        def owait(out_v, osem):
            pltpu.make_async_copy(
                out_hbm.at[pl.ds(qbase, CQ)], out_v, osem).wait()

        start(0, rows_a, sem_a)
        start(1, rows_b, sem_b)

        def pair(k, carry):
            c0 = 2 * k
            @pl.when(lax.rem(k, 4) == 0)
            def _():
                pltpu.sync_copy(
                    w_hbm.at[pl.ds((qbase + c0 * CQ) * 48, 8 * CQ * 48)], w_v)

            woff = lax.rem(c0, 8) * (CQ * 48)
            pltpu.make_async_copy(
                f2_hbm.at[idx_v.at[pl.ds(0, CQ * 3)]], rows_a, sem_a).wait()

            @pl.when(k > 0)
            def _():
                owait(out_a, osem_a)

            compute(rows_a, out_a, woff, c0)
            pltpu.async_copy(out_a, out_hbm.at[pl.ds(qbase + c0 * CQ, CQ)],
                             osem_a)

            @pl.when(k + 1 < npair)
            def _():
                start(c0 + 2, rows_a, sem_a)

            pltpu.make_async_copy(
                f2_hbm.at[idx_v.at[pl.ds(0, CQ * 3)]], rows_b, sem_b).wait()

            @pl.when(k > 0)
            def _():
                owait(out_b, osem_b)

            compute(rows_b, out_b, woff + CQ * 48, c0 + 1)
            pltpu.async_copy(out_b, out_hbm.at[pl.ds(qbase + (c0 + 1) * CQ, CQ)],
                             osem_b)

            @pl.when(k + 1 < npair)
            def _():
                start(c0 + 3, rows_b, sem_b)

            return carry

        lax.fori_loop(0, npair, pair, 0)
        owait(out_a, osem_a)
        owait(out_b, osem_b)

    return sc_interp


def _k2_body(h1_ref, s1_ref, ss1_ref, g0_ref, bt0_ref, w1t_ref, b1_ref,
             h2_ref, s2_ref, ss2_ref, *, count):
    mean = s1_ref[...] / count
    var = ss1_ref[...] / count - mean * mean
    rstd = jax.lax.rsqrt(var + 1e-5)
    a1 = jnp.maximum((h1_ref[0] - mean) * (rstd * g0_ref[...]) + bt0_ref[...],
                     0.0)
    h2 = _bdot(a1, w1t_ref[...]) + b1_ref[...]
    h2_ref[0] = h2

    @pl.when((pl.program_id(0) == 0) & (pl.program_id(1) == 0))
    def _():
        s2_ref[...] = jnp.zeros_like(s2_ref)
        ss2_ref[...] = jnp.zeros_like(ss2_ref)

    s2_ref[...] += jnp.sum(h2, axis=0, keepdims=True)
    ss2_ref[...] += jnp.sum(h2 * h2, axis=0, keepdims=True)


def _k3_body(h2_ref, s2_ref, ss2_ref, g1_ref, bt1_ref, out_ref, *, count):
    mean = s2_ref[...] / count
    var = ss2_ref[...] / count - mean * mean
    rstd = jax.lax.rsqrt(var + 1e-5)
    out_ref[0] = jnp.maximum(
        (h2_ref[0] - mean) * (rstd * g1_ref[...]) + bt1_ref[...], 0.0)


def kernel(x1, x2, feaure1, feature2, W0, b0, gamma0, beta0, W1, b1, gamma1, beta1):
    B, N1, _ = x1.shape
    N2 = x2.shape[1]
    C1 = feaure1.shape[-1]
    C2 = feature2.shape[-1]
    H1 = W0.shape[0]
    H2 = W1.shape[0]
    count = float(B * N1)

    TQ1 = min(256, N1)
    TQ2 = min(512, N1)
    TQ3 = min(1024, N1)
    nb1 = N1 // TQ1
    nb2 = N1 // TQ2
    nb3 = N1 // TQ3

    x2t = jnp.swapaxes(x2, 1, 2)
    w0t = jnp.transpose(W0)
    w0at, w0bt = w0t[:C1], w0t[C1:]
    w1t = jnp.transpose(W1)
    b0r = b0.reshape(1, H1)
    g0r = gamma0.reshape(1, H1)
    bt0r = beta0.reshape(1, H1)
    b1r = b1.reshape(1, H2)
    g1r = gamma1.reshape(1, H2)
    bt1r = beta1.reshape(1, H2)

    rep = lambda shape: pl.BlockSpec(shape, lambda b, i: (0,) * len(shape))
    per_b = lambda shape: pl.BlockSpec(shape, lambda b, i: (b, 0, 0))
    tiled = lambda shape: pl.BlockSpec(shape, lambda b, i: (b, i, 0))

    gidx, wgt = pl.pallas_call(
        functools.partial(_k1_body, n2=N2),
        grid=(B, nb1),
        in_specs=[
            tiled((1, TQ1, 3)),
            per_b((1, 3, N2)),
        ],
        out_specs=[
            tiled((1, TQ1, 3)),
            tiled((1, TQ1, 48)),
        ],
        out_shape=[
            jax.ShapeDtypeStruct((B, N1, 3), jnp.int32),
            jax.ShapeDtypeStruct((B, N1, 48), jnp.float32),
        ],
    )(x1, x2t)

    interp = _make_sc_interp(B * N1, B * N2, C2)(
        feature2.reshape(B * N2, C2),
        gidx.reshape(B * N1 * 3),
        wgt.reshape(B * N1 * 48),
    ).reshape(B, N1, C2)

    h1, s1, ss1 = pl.pallas_call(
        _k1b_body,
        grid=(B, nb1),
        in_specs=[
            tiled((1, TQ1, C1)),
            tiled((1, TQ1, C2)),
            rep((C1, H1)),
            rep((C2, H1)),
            rep((1, H1)),
        ],
        out_specs=[
            tiled((1, TQ1, H1)),
            rep((1, H1)),
            rep((1, H1)),
        ],
        out_shape=[
            jax.ShapeDtypeStruct((B, N1, H1), jnp.float32),
            jax.ShapeDtypeStruct((1, H1), jnp.float32),
            jax.ShapeDtypeStruct((1, H1), jnp.float32),
        ],
    )(feaure1, interp, w0at, w0bt, b0r)

    h2, s2, ss2 = pl.pallas_call(
        functools.partial(_k2_body, count=count),
        grid=(B, nb2),
        in_specs=[
            tiled((1, TQ2, H1)),
            rep((1, H1)),
            rep((1, H1)),
            rep((1, H1)),
            rep((1, H1)),
            rep((H1, H2)),
            rep((1, H2)),
        ],
        out_specs=[
            tiled((1, TQ2, H2)),
            rep((1, H2)),
            rep((1, H2)),
        ],
        out_shape=[
            jax.ShapeDtypeStruct((B, N1, H2), jnp.float32),
            jax.ShapeDtypeStruct((1, H2), jnp.float32),
            jax.ShapeDtypeStruct((1, H2), jnp.float32),
        ],
    )(h1, s1, ss1, g0r, bt0r, w1t, b1r)

    out = pl.pallas_call(
        functools.partial(_k3_body, count=count),
        grid=(B, nb3),
        in_specs=[
            tiled((1, TQ3, H2)),
            rep((1, H2)),
            rep((1, H2)),
            rep((1, H2)),
            rep((1, H2)),
        ],
        out_specs=tiled((1, TQ3, H2)),
        out_shape=jax.ShapeDtypeStruct((B, N1, H2), jnp.float32),
    )(h2, s2, ss2, g1r, bt1r)

    return out
